# baseline (speedup 1.0000x reference)
import functools

import jax
import jax.numpy as jnp
from jax import lax
from jax.experimental import pallas as pl
from jax.experimental.pallas import tpu as pltpu
from jax.experimental.pallas import tpu_sc as plsc

N = 10000
E = 160000
D = 256
R = 16
H = 128
NR = N * R
NSUB = 16
EP = E // NSUB
G = 80
CH = EP // G
BN = 1000
NB = N // BN


def _transform_body(apply_scale, h_ref, scale_ref, w_ref, rf_ref, out_ref):
    h = h_ref[...]
    if apply_scale:
        h = h * scale_ref[...]
    for r in range(R):
        p = jnp.dot(h, w_ref[r], preferred_element_type=jnp.float32)
        p = p + rf_ref[r][None, :]
        out_ref[0, :, r * H:(r + 1) * H] = p[:, :H]
        out_ref[1, :, r * H:(r + 1) * H] = p[:, H:]


def _transform(h, scale, w, rf, apply_scale):
    return pl.pallas_call(
        functools.partial(_transform_body, apply_scale),
        grid=(NB,),
        in_specs=[
            pl.BlockSpec((BN, D), lambda i: (i, 0)),
            pl.BlockSpec((BN, 1), lambda i: (i, 0)),
            pl.BlockSpec((R, D, D), lambda i: (0, 0, 0)),
            pl.BlockSpec((R, D), lambda i: (0, 0)),
        ],
        out_specs=pl.BlockSpec((2, BN, R * H), lambda i: (0, i, 0)),
        out_shape=jax.ShapeDtypeStruct((2, N, R * H), jnp.float32),
    )(h, scale, w, rf)


@functools.cache
def _make_sc_scatter():
    mesh = plsc.VectorSubcoreMesh(core_axis_name="c", subcore_axis_name="s")
    return functools.partial(
        pl.kernel,
        mesh=mesh,
        out_type=jax.ShapeDtypeStruct((N, D), jnp.float32),
        scratch_types=[
            pltpu.VMEM((EP,), jnp.int32),
            pltpu.VMEM((EP,), jnp.int32),
            pltpu.VMEM((G, H), jnp.float32),
            pltpu.VMEM((G, H), jnp.float32),
            pltpu.VMEM((G,), jnp.int32),
            pltpu.VMEM((G,), jnp.int32),
            pltpu.VMEM((G,), jnp.int32),
            pltpu.VMEM((G,), jnp.int32),
            pltpu.VMEM_SHARED((N, H), jnp.float32),
            pltpu.SemaphoreType.DMA,
            pltpu.SemaphoreType.DMA,
        ],
    )(_sc_scatter_body)


def _sc_scatter_body(key_hbm, dst_hbm, t_hbm, out_hbm,
                     key_v, dst_v, rows0, rows1, kb0, kb1, db0, db1,
                     acc, sem0, sem1):
    c = lax.axis_index("c")
    s = lax.axis_index("s")
    base = s * EP
    pltpu.sync_copy(key_hbm.at[pl.ds(base, EP)], key_v)
    pltpu.sync_copy(dst_hbm.at[pl.ds(base, EP)], dst_v)

    def _zero(i, carry):
        r = i // (H // 16)
        col = (i % (H // 16)) * 16
        rows0[r, pl.ds(col, 16)] = jnp.zeros((16,), jnp.float32)
        return carry
    lax.fori_loop(0, G * (H // 16), _zero, 0)

    zbase = s * (N // NSUB)
    for k in range(7):
        pltpu.sync_copy(rows0, acc.at[pl.ds(zbase + k * G, G)])
    pltpu.sync_copy(rows0.at[pl.ds(0, 65)], acc.at[pl.ds(zbase + 7 * G, 65)])
    plsc.subcore_barrier()

    koff = c * NR

    def _stage(chunk, kb, db):
        cb = chunk * G
        def _cp(j, inner):
            sl = pl.ds(j * 16, 16)
            kb[sl] = key_v[pl.ds(cb + j * 16, 16)] + koff
            db[sl] = dst_v[pl.ds(cb + j * 16, 16)]
            return inner
        lax.fori_loop(0, G // 16, _cp, 0)

    _stage(0, kb0, db0)
    g0 = pltpu.async_copy(t_hbm.at[kb0], rows0, sem0)

    def _pair(g, carry):
        a = 2 * g
        _stage(a + 1, kb1, db1)
        g1 = pltpu.async_copy(t_hbm.at[kb1], rows1, sem1)
        pltpu.make_async_copy(t_hbm.at[kb0], rows0, sem0).wait()
        pltpu.sync_copy(rows0, acc.at[db0], add=True)
        _stage(a + 2, kb0, db0)
        pltpu.async_copy(t_hbm.at[kb0], rows0, sem0)
        pltpu.make_async_copy(t_hbm.at[kb1], rows1, sem1).wait()
        pltpu.sync_copy(rows1, acc.at[db1], add=True)
        return carry
    lax.fori_loop(0, (CH - 1) // 2, _pair, 0)

    pltpu.make_async_copy(t_hbm.at[kb0], rows0, sem0).wait()
    pltpu.sync_copy(rows0, acc.at[db0], add=True)
    plsc.subcore_barrier()

    rbase = s * 624
    pltpu.sync_copy(acc.at[pl.ds(rbase, 624)],
                    out_hbm.at[pl.ds(rbase, 624), pl.ds(c * H, H)])
    @pl.when(s == NSUB - 1)
    def _tail():
        pltpu.sync_copy(acc.at[pl.ds(9984, 16)],
                        out_hbm.at[pl.ds(9984, 16), pl.ds(c * H, H)])


def _final_body(agg_ref, norm_ref, lw_ref, out_ref):
    h2 = agg_ref[...] * norm_ref[...]
    out_ref[...] = jnp.maximum(
        h2 + jnp.dot(h2, lw_ref[...], preferred_element_type=jnp.float32), 0.0)


def _final(agg, normv, loop_weight):
    return pl.pallas_call(
        _final_body,
        grid=(NB,),
        in_specs=[
            pl.BlockSpec((BN, D), lambda i: (i, 0)),
            pl.BlockSpec((BN, 1), lambda i: (i, 0)),
            pl.BlockSpec((D, D), lambda i: (0, 0)),
        ],
        out_specs=pl.BlockSpec((BN, D), lambda i: (i, 0)),
        out_shape=jax.ShapeDtypeStruct((N, D), jnp.float32),
    )(agg, normv, loop_weight)


def kernel(x, norm, edge_index, edge_type, rFeatures, relation_weights, loop_weight):
    src = edge_index[0].astype(jnp.int32)
    dst = edge_index[1].astype(jnp.int32)
    key = src * R + edge_type.astype(jnp.int32)
    normv = norm.reshape(N, 1)

    sc_scatter = _make_sc_scatter()
    t0 = _transform(x, normv, relation_weights, rFeatures, apply_scale=False)
    agg1 = sc_scatter(key, dst, t0.reshape(2 * NR, H))
    t1 = _transform(agg1, normv, relation_weights, rFeatures, apply_scale=True)
    agg2 = sc_scatter(key, dst, t1.reshape(2 * NR, H))
    return _final(agg2, normv, loop_weight)


# speedup vs baseline: 1.8514x; 1.8514x over previous
import functools

import jax
import jax.numpy as jnp
from jax import lax
from jax.experimental import pallas as pl
from jax.experimental.pallas import tpu as pltpu
from jax.experimental.pallas import tpu_sc as plsc

N = 10000
E = 160000
D = 256
R = 16
H = 128
NR = N * R
NSUB = 16
EP = E // NSUB
G = 80
CH = EP // G
BN = 1000
NB = N // BN


def _transform_body(apply_scale, h_ref, scale_ref, w_ref, rf_ref, out_ref):
    h = h_ref[...]
    if apply_scale:
        h = h * scale_ref[...]
    for r in range(R):
        p = jnp.dot(h, w_ref[r], preferred_element_type=jnp.float32)
        p = p + rf_ref[r][None, :]
        out_ref[0, r] = p[:, :H]
        out_ref[1, r] = p[:, H:]


def _transform(h, scale, w, rf, apply_scale):
    return pl.pallas_call(
        functools.partial(_transform_body, apply_scale),
        grid=(NB,),
        in_specs=[
            pl.BlockSpec((BN, D), lambda i: (i, 0)),
            pl.BlockSpec((BN, 1), lambda i: (i, 0)),
            pl.BlockSpec((R, D, D), lambda i: (0, 0, 0)),
            pl.BlockSpec((R, D), lambda i: (0, 0)),
        ],
        out_specs=pl.BlockSpec((2, R, BN, H), lambda i: (0, 0, i, 0)),
        out_shape=jax.ShapeDtypeStruct((2, R, N, H), jnp.float32),
    )(h, scale, w, rf)


@functools.cache
def _make_sc_scatter():
    mesh = plsc.VectorSubcoreMesh(core_axis_name="c", subcore_axis_name="s")
    return functools.partial(
        pl.kernel,
        mesh=mesh,
        out_type=jax.ShapeDtypeStruct((N, D), jnp.float32),
        scratch_types=[
            pltpu.VMEM((EP,), jnp.int32),
            pltpu.VMEM((EP,), jnp.int32),
            pltpu.VMEM((G, H), jnp.float32),
            pltpu.VMEM((G, H), jnp.float32),
            pltpu.VMEM((G,), jnp.int32),
            pltpu.VMEM((G,), jnp.int32),
            pltpu.VMEM((G,), jnp.int32),
            pltpu.VMEM((G,), jnp.int32),
            pltpu.VMEM_SHARED((N, H), jnp.float32),
            pltpu.SemaphoreType.DMA,
            pltpu.SemaphoreType.DMA,
        ],
    )(_sc_scatter_body)


def _sc_scatter_body(key_hbm, dst_hbm, t_hbm, out_hbm,
                     key_v, dst_v, rows0, rows1, kb0, kb1, db0, db1,
                     acc, sem0, sem1):
    c = lax.axis_index("c")
    s = lax.axis_index("s")
    base = s * EP
    pltpu.sync_copy(key_hbm.at[pl.ds(base, EP)], key_v)
    pltpu.sync_copy(dst_hbm.at[pl.ds(base, EP)], dst_v)

    def _zero(i, carry):
        r = i // (H // 16)
        col = (i % (H // 16)) * 16
        rows0[r, pl.ds(col, 16)] = jnp.zeros((16,), jnp.float32)
        return carry
    lax.fori_loop(0, G * (H // 16), _zero, 0)

    zbase = s * (N // NSUB)
    for k in range(7):
        pltpu.sync_copy(rows0, acc.at[pl.ds(zbase + k * G, G)])
    pltpu.sync_copy(rows0.at[pl.ds(0, 65)], acc.at[pl.ds(zbase + 7 * G, 65)])
    plsc.subcore_barrier()

    koff = c * NR

    def _stage(chunk, kb, db):
        cb = chunk * G
        def _cp(j, inner):
            sl = pl.ds(j * 16, 16)
            kb[sl] = key_v[pl.ds(cb + j * 16, 16)] + koff
            db[sl] = dst_v[pl.ds(cb + j * 16, 16)]
            return inner
        lax.fori_loop(0, G // 16, _cp, 0)

    _stage(0, kb0, db0)
    g0 = pltpu.async_copy(t_hbm.at[kb0], rows0, sem0)

    def _pair(g, carry):
        a = 2 * g
        _stage(a + 1, kb1, db1)
        g1 = pltpu.async_copy(t_hbm.at[kb1], rows1, sem1)
        pltpu.make_async_copy(t_hbm.at[kb0], rows0, sem0).wait()
        pltpu.sync_copy(rows0, acc.at[db0], add=True)
        _stage(a + 2, kb0, db0)
        pltpu.async_copy(t_hbm.at[kb0], rows0, sem0)
        pltpu.make_async_copy(t_hbm.at[kb1], rows1, sem1).wait()
        pltpu.sync_copy(rows1, acc.at[db1], add=True)
        return carry
    lax.fori_loop(0, (CH - 1) // 2, _pair, 0)

    pltpu.make_async_copy(t_hbm.at[kb0], rows0, sem0).wait()
    pltpu.sync_copy(rows0, acc.at[db0], add=True)
    plsc.subcore_barrier()

    rbase = s * 624
    pltpu.sync_copy(acc.at[pl.ds(rbase, 624)],
                    out_hbm.at[pl.ds(rbase, 624), pl.ds(c * H, H)])
    @pl.when(s == NSUB - 1)
    def _tail():
        pltpu.sync_copy(acc.at[pl.ds(9984, 16)],
                        out_hbm.at[pl.ds(9984, 16), pl.ds(c * H, H)])


def _final_body(agg_ref, norm_ref, lw_ref, out_ref):
    h2 = agg_ref[...] * norm_ref[...]
    out_ref[...] = jnp.maximum(
        h2 + jnp.dot(h2, lw_ref[...], preferred_element_type=jnp.float32), 0.0)


def _final(agg, normv, loop_weight):
    return pl.pallas_call(
        _final_body,
        grid=(NB,),
        in_specs=[
            pl.BlockSpec((BN, D), lambda i: (i, 0)),
            pl.BlockSpec((BN, 1), lambda i: (i, 0)),
            pl.BlockSpec((D, D), lambda i: (0, 0)),
        ],
        out_specs=pl.BlockSpec((BN, D), lambda i: (i, 0)),
        out_shape=jax.ShapeDtypeStruct((N, D), jnp.float32),
    )(agg, normv, loop_weight)


def kernel(x, norm, edge_index, edge_type, rFeatures, relation_weights, loop_weight):
    src = edge_index[0].astype(jnp.int32)
    dst = edge_index[1].astype(jnp.int32)
    key = edge_type.astype(jnp.int32) * N + src
    normv = norm.reshape(N, 1)

    sc_scatter = _make_sc_scatter()
    t0 = _transform(x, normv, relation_weights, rFeatures, apply_scale=False)
    agg1 = sc_scatter(key, dst, t0.reshape(2 * NR, H))
    t1 = _transform(agg1, normv, relation_weights, rFeatures, apply_scale=True)
    agg2 = sc_scatter(key, dst, t1.reshape(2 * NR, H))
    return _final(agg2, normv, loop_weight)
